# baseline (speedup 1.0000x reference)
import functools

import jax
import jax.numpy as jnp
from jax import lax
from jax.experimental import pallas as pl
from jax.experimental.pallas import tpu as pltpu
from jax.experimental.pallas import tpu_sc as plsc

N = 100000
K = 64
TAU = 1.0
B = 1024
NP = 102400
NBLK = NP // B
R2 = NP // 128
PB = 80
NW = 32
CROWS = 8
NCHUNK = R2 // CROWS


def _dense_body(hard_ref, logits_ref, u_ref, tril_ref,
                soft_ref, labels_ref, rank_ref, counts_ref, carry_ref):
    pid = pl.program_id(0)

    @pl.when(pid == 0)
    def _():
        carry_ref[...] = jnp.zeros((1, K), jnp.float32)

    z = logits_ref[...] + (-jnp.log(-jnp.log(u_ref[...])))
    z = z / TAU
    m = jnp.max(z, axis=1, keepdims=True)
    e = jnp.exp(z - m)
    s = jnp.sum(e, axis=1, keepdims=True)
    y = e / s

    kiota = lax.broadcasted_iota(jnp.int32, (B, K), 1)
    ymax = jnp.max(y, axis=1, keepdims=True)
    labels = jnp.min(jnp.where(y == ymax, kiota, K), axis=1)

    oh_bool = kiota == labels[:, None]
    y_hard = jnp.where(oh_bool, 1.0, 0.0).astype(jnp.float32)
    h = hard_ref[0, 0]
    soft_ref[...] = jnp.where(h != 0, (y_hard - y) + y, y)

    row = pid * B + lax.broadcasted_iota(jnp.int32, (B, 1), 0)
    valid = row < N
    ohm = jnp.where(oh_bool & valid, 1.0, 0.0).astype(jnp.float32)

    ranks_in = jnp.dot(tril_ref[...], ohm.astype(jnp.bfloat16),
                       preferred_element_type=jnp.float32)
    carry = carry_ref[...]
    rank = jnp.sum((carry + ranks_in) * ohm, axis=1)
    carry_new = carry + jnp.sum(ohm, axis=0, keepdims=True)
    carry_ref[...] = carry_new

    labels_ref[...] = labels.reshape(8, 128)
    rank_ref[...] = rank.astype(jnp.int32).reshape(8, 128)

    @pl.when(pid == NBLK - 1)
    def _():
        counts_ref[...] = carry_new.astype(jnp.int32)


def _pos_body(counts_ref, labels_ref, rank_ref, pos_ref, base_ref):
    @pl.when(pl.program_id(0) == 0)
    def _():
        def body(k, acc):
            base_ref[k] = acc
            return acc + counts_ref[0, k]
        lax.fori_loop(0, K, body, 0)

    lab = labels_ref[...]
    acc = jnp.zeros((PB, 128), jnp.int32)
    for k in range(K):
        acc = jnp.where(lab == k, base_ref[k], acc)
    flat = (pl.program_id(0) * (PB * 128)
            + lax.broadcasted_iota(jnp.int32, (PB, 128), 0) * 128
            + lax.broadcasted_iota(jnp.int32, (PB, 128), 1))
    pos = acc + rank_ref[...]
    pos_ref[...] = jnp.where(flat < N, pos, flat)


@functools.cache
def _make_scatter():
    mesh = plsc.VectorSubcoreMesh(core_axis_name="c", subcore_axis_name="s")

    @functools.partial(
        pl.kernel,
        mesh=mesh,
        out_type=jax.ShapeDtypeStruct((NP,), jnp.int32),
        scratch_types=[
            pltpu.VMEM((CROWS, 128), jnp.int32),
            pltpu.VMEM((CROWS, 128), jnp.int32),
            pltpu.SemaphoreType.DMA,
            pltpu.SemaphoreType.DMA,
        ],
    )
    def scatter_kernel(pos_hbm, order_hbm, pos_v, vals_v, sem_in, sem_out):
        wid = lax.axis_index("s") * 2 + lax.axis_index("c")
        nch = jnp.where(wid < NCHUNK - (NCHUNK // NW) * NW, NCHUNK // NW + 1,
                        NCHUNK // NW)

        @pl.loop(0, nch)
        def _(i):
            chunk = wid + i * NW
            r0 = pl.multiple_of(chunk * CROWS, CROWS)
            pltpu.async_copy(pos_hbm.at[pl.ds(r0, CROWS)], pos_v,
                             sem_in).wait()
            for r in range(CROWS):
                for j in range(8):
                    vals_v[r, pl.ds(j * 16, 16)] = (
                        lax.iota(jnp.int32, 16)
                        + ((r0 + r) * 128 + j * 16))
            handles = [
                pltpu.async_copy(vals_v.at[r], order_hbm.at[pos_v.at[r]],
                                 sem_out)
                for r in range(CROWS)
            ]
            for hd in handles:
                hd.wait()

    return scatter_kernel


def kernel(logits, u, hard):
    pad = NP - N
    logits_p = jnp.pad(logits, ((0, pad), (0, 0)))
    u_p = jnp.pad(u, ((0, pad), (0, 0)), constant_values=0.5)
    hard_s = jnp.asarray(hard, jnp.int32).reshape(1, 1)
    tril = jnp.tril(jnp.ones((B, B), jnp.bfloat16), -1)

    soft_p, labels2d, rank2d, counts2d = pl.pallas_call(
        _dense_body,
        grid=(NBLK,),
        in_specs=[
            pl.BlockSpec(memory_space=pltpu.SMEM),
            pl.BlockSpec((B, K), lambda i: (i, 0)),
            pl.BlockSpec((B, K), lambda i: (i, 0)),
            pl.BlockSpec((B, B), lambda i: (0, 0)),
        ],
        out_specs=[
            pl.BlockSpec((B, K), lambda i: (i, 0)),
            pl.BlockSpec((8, 128), lambda i: (i, 0)),
            pl.BlockSpec((8, 128), lambda i: (i, 0)),
            pl.BlockSpec((1, K), lambda i: (0, 0)),
        ],
        out_shape=[
            jax.ShapeDtypeStruct((NP, K), jnp.float32),
            jax.ShapeDtypeStruct((R2, 128), jnp.int32),
            jax.ShapeDtypeStruct((R2, 128), jnp.int32),
            jax.ShapeDtypeStruct((1, K), jnp.int32),
        ],
        scratch_shapes=[pltpu.VMEM((1, K), jnp.float32)],
    )(hard_s, logits_p, u_p, tril)

    pos2d = pl.pallas_call(
        _pos_body,
        grid=(R2 // PB,),
        in_specs=[
            pl.BlockSpec(memory_space=pltpu.SMEM),
            pl.BlockSpec((PB, 128), lambda i: (i, 0)),
            pl.BlockSpec((PB, 128), lambda i: (i, 0)),
        ],
        out_specs=pl.BlockSpec((PB, 128), lambda i: (i, 0)),
        out_shape=jax.ShapeDtypeStruct((R2, 128), jnp.int32),
        scratch_shapes=[pltpu.SMEM((K,), jnp.int32)],
    )(counts2d, labels2d, rank2d)

    order_p = _make_scatter()(pos2d)

    order = order_p[:N]
    counts = counts2d.reshape(K)
    partition_labels = labels2d.reshape(NP)[:N]
    soft = soft_p[:N]
    return (order, counts, partition_labels, soft)


# speedup vs baseline: 1.4794x; 1.4794x over previous
import functools

import jax
import jax.numpy as jnp
from jax import lax
from jax.experimental import pallas as pl
from jax.experimental.pallas import tpu as pltpu
from jax.experimental.pallas import tpu_sc as plsc

N = 100000
K = 64
TAU = 1.0
B = 1024
NP = 102400
NBLK = NP // B
R2 = NP // 128
PB = 80
NW = 32
H = NP // 2
PSUB = NP // 16
GROUPS = PSUB // 128
TRASH = 128
SHARE = H // 16


def _dense_body(hard_ref, logits_ref, u_ref, tril_ref,
                soft_ref, labels_ref, rank_ref, counts_ref, carry_ref):
    pid = pl.program_id(0)

    @pl.when(pid == 0)
    def _():
        carry_ref[...] = jnp.zeros((1, K), jnp.float32)

    z = logits_ref[...] + (-jnp.log(-jnp.log(u_ref[...])))
    z = z / TAU
    m = jnp.max(z, axis=1, keepdims=True)
    e = jnp.exp(z - m)
    s = jnp.sum(e, axis=1, keepdims=True)
    y = e / s

    kiota = lax.broadcasted_iota(jnp.int32, (B, K), 1)
    ymax = jnp.max(y, axis=1, keepdims=True)
    labels = jnp.min(jnp.where(y == ymax, kiota, K), axis=1)

    oh_bool = kiota == labels[:, None]
    y_hard = jnp.where(oh_bool, 1.0, 0.0).astype(jnp.float32)
    h = hard_ref[0, 0]
    soft_ref[...] = jnp.where(h != 0, (y_hard - y) + y, y)

    row = pid * B + lax.broadcasted_iota(jnp.int32, (B, 1), 0)
    valid = row < N
    ohm = jnp.where(oh_bool & valid, 1.0, 0.0).astype(jnp.float32)

    ranks_in = jnp.dot(tril_ref[...], ohm.astype(jnp.bfloat16),
                       preferred_element_type=jnp.float32)
    carry = carry_ref[...]
    rank = jnp.sum((carry + ranks_in) * ohm, axis=1)
    carry_new = carry + jnp.sum(ohm, axis=0, keepdims=True)
    carry_ref[...] = carry_new

    labels_ref[...] = labels.reshape(8, 128)
    rank_ref[...] = rank.astype(jnp.int32).reshape(8, 128)

    @pl.when(pid == NBLK - 1)
    def _():
        counts_ref[...] = carry_new.astype(jnp.int32)


def _pos_body(counts_ref, labels_ref, rank_ref, pos_ref, base_ref):
    @pl.when(pl.program_id(0) == 0)
    def _():
        def body(k, acc):
            base_ref[k] = acc
            return acc + counts_ref[0, k]
        lax.fori_loop(0, K, body, 0)

    lab = labels_ref[...]
    acc = jnp.zeros((PB, 128), jnp.int32)
    for k in range(K):
        acc = jnp.where(lab == k, base_ref[k], acc)
    flat = (pl.program_id(0) * (PB * 128)
            + lax.broadcasted_iota(jnp.int32, (PB, 128), 0) * 128
            + lax.broadcasted_iota(jnp.int32, (PB, 128), 1))
    pos = acc + rank_ref[...]
    pos_ref[...] = jnp.where(flat < N, pos, flat).reshape(PB * 128)


@functools.cache
def _make_scatter():
    mesh = plsc.VectorSubcoreMesh(core_axis_name="c", subcore_axis_name="s")

    @functools.partial(
        pl.kernel,
        mesh=mesh,
        out_type=jax.ShapeDtypeStruct((NP,), jnp.int32),
        scratch_types=[
            pltpu.VMEM((PSUB,), jnp.int32),
            pltpu.VMEM((GROUPS, 128), jnp.int32),
            pltpu.VMEM((GROUPS, 128), jnp.int32),
            pltpu.VMEM_SHARED((H + TRASH,), jnp.int32),
            pltpu.SemaphoreType.DMA,
            pltpu.SemaphoreType.DMA,
        ],
    )
    def scatter_kernel(pos_hbm, order_hbm, pos_v, idx_v, vals_v, shared,
                       sem_in, sem_out):
        cid = lax.axis_index("c")
        sid = lax.axis_index("s")
        base = sid * PSUB
        pltpu.async_copy(pos_hbm.at[pl.ds(base, PSUB)], pos_v, sem_in).wait()
        half0 = cid * H

        @pl.loop(0, GROUPS)
        def _(g):
            for jj in range(8):
                off = pl.multiple_of(g * 128 + jj * 16, 16)
                p = pos_v[pl.ds(off, 16)]
                loc = p - half0
                ok = (loc >= 0) & (loc < H)
                trash = H + lax.iota(jnp.int32, 16) + jj * 16
                idx_v[g, pl.ds(jj * 16, 16)] = jnp.where(ok, loc, trash)
                vals_v[g, pl.ds(jj * 16, 16)] = (
                    lax.iota(jnp.int32, 16) + (base + g * 128 + jj * 16))

        handles = [
            pltpu.async_copy(vals_v.at[g], shared.at[idx_v.at[g]], sem_out)
            for g in range(GROUPS)
        ]
        for hd in handles:
            hd.wait()
        plsc.subcore_barrier()
        pltpu.sync_copy(shared.at[pl.ds(sid * SHARE, SHARE)],
                        order_hbm.at[pl.ds(half0 + sid * SHARE, SHARE)])

    return scatter_kernel


def kernel(logits, u, hard):
    pad = NP - N
    logits_p = jnp.pad(logits, ((0, pad), (0, 0)))
    u_p = jnp.pad(u, ((0, pad), (0, 0)), constant_values=0.5)
    hard_s = jnp.asarray(hard, jnp.int32).reshape(1, 1)
    tril = jnp.tril(jnp.ones((B, B), jnp.bfloat16), -1)

    soft_p, labels2d, rank2d, counts2d = pl.pallas_call(
        _dense_body,
        grid=(NBLK,),
        in_specs=[
            pl.BlockSpec(memory_space=pltpu.SMEM),
            pl.BlockSpec((B, K), lambda i: (i, 0)),
            pl.BlockSpec((B, K), lambda i: (i, 0)),
            pl.BlockSpec((B, B), lambda i: (0, 0)),
        ],
        out_specs=[
            pl.BlockSpec((B, K), lambda i: (i, 0)),
            pl.BlockSpec((8, 128), lambda i: (i, 0)),
            pl.BlockSpec((8, 128), lambda i: (i, 0)),
            pl.BlockSpec((1, K), lambda i: (0, 0)),
        ],
        out_shape=[
            jax.ShapeDtypeStruct((NP, K), jnp.float32),
            jax.ShapeDtypeStruct((R2, 128), jnp.int32),
            jax.ShapeDtypeStruct((R2, 128), jnp.int32),
            jax.ShapeDtypeStruct((1, K), jnp.int32),
        ],
        scratch_shapes=[pltpu.VMEM((1, K), jnp.float32)],
    )(hard_s, logits_p, u_p, tril)

    pos1d = pl.pallas_call(
        _pos_body,
        grid=(R2 // PB,),
        in_specs=[
            pl.BlockSpec(memory_space=pltpu.SMEM),
            pl.BlockSpec((PB, 128), lambda i: (i, 0)),
            pl.BlockSpec((PB, 128), lambda i: (i, 0)),
        ],
        out_specs=pl.BlockSpec((PB * 128,), lambda i: (i,)),
        out_shape=jax.ShapeDtypeStruct((NP,), jnp.int32),
        scratch_shapes=[pltpu.SMEM((K,), jnp.int32)],
    )(counts2d, labels2d, rank2d)

    order_p = _make_scatter()(pos1d)

    order = order_p[:N]
    counts = counts2d.reshape(K)
    partition_labels = labels2d.reshape(NP)[:N]
    soft = soft_p[:N]
    return (order, counts, partition_labels, soft)
